# baseline (speedup 1.0000x reference)
import functools
import numpy as np
import jax
import jax.numpy as jnp
from jax import lax
from jax.experimental import pallas as pl
from jax.experimental.pallas import tpu as pltpu
from jax.experimental.pallas import tpu_sc as plsc

_STRIDES = [8, 16, 32, 64, 128]
_LIMITS = [(-1.0, 64.0), (64.0, 128.0), (128.0, 256.0), (256.0, 512.0),
           (512.0, 99999999.0)]
_FEAT = [(64, 64), (32, 32), (16, 16), (8, 8), (4, 4)]
_B, _M = 8, 64
_N = sum(h * w for h, w in _FEAT)
_NPAD = 5504
_QLOC = _NPAD // 4
_NG = _QLOC // 16
_BIG = 99999999.0


def _location_tables():
    xs = np.zeros(_NPAD, np.float32)
    ys = np.zeros(_NPAD, np.float32)
    llo = np.full(_NPAD, 1e9, np.float32)
    lhi = np.full(_NPAD, -1e9, np.float32)
    rad = np.full(_NPAD, -1.0, np.float32)
    o = 0
    for (h, w), s, (lo, hi) in zip(_FEAT, _STRIDES, _LIMITS):
        sx = np.arange(0, w * s, s, dtype=np.float32) + s // 2
        sy = np.arange(0, h * s, s, dtype=np.float32) + s // 2
        yy, xx = np.meshgrid(sy, sx, indexing='ij')
        n = h * w
        xs[o:o + n] = xx.reshape(-1)
        ys[o:o + n] = yy.reshape(-1)
        llo[o:o + n] = lo
        lhi[o:o + n] = hi
        rad[o:o + n] = s * 1.5
        o += n
    return xs, ys, llo, lhi, rad


_XS, _YS, _LLO, _LHI, _RAD = _location_tables()


def _splat(v, j, dtype=jnp.float32):
    return jnp.full((16,), v[j], dtype)


def _dyn_gather(v, iv):
    return lax.gather(
        v, iv[:, None],
        dimension_numbers=lax.GatherDimensionNumbers(
            offset_dims=(), collapsed_slice_dims=(0,), start_index_map=(0,)),
        slice_sizes=(1,),
        mode=lax.GatherScatterMode.PROMISE_IN_BOUNDS)


def _sqrt16(x):
    i = lax.bitcast_convert_type(x, jnp.int32)
    y = lax.bitcast_convert_type(jnp.int32(0x5F3759DF) - (i >> 1), jnp.float32)
    for _ in range(3):
        y = y * (1.5 - 0.5 * x * y * y)
    return x * y


def _sc_body(xs_h, ys_h, llo_h, lhi_h, rad_h, bx1_h, by1_h, bx2_h, by2_h,
             lab_h, cls_o, ctr_o, l_o, t_o, r_o, b_o,
             xs_v, ys_v, llo_v, lhi_v, rad_v,
             bx1_v, by1_v, bx2_v, by2_v, lab_v,
             tx1_v, ty1_v, tx2_v, ty2_v, tcx_v, tcy_v, tidx_v,
             cls_v, ctr_v, l_v, t_v, r_v, b_v):
    wid = lax.axis_index("s") * 2 + lax.axis_index("c")
    b = wid // 4
    q = wid % 4
    loc0 = q * _QLOC
    box0 = b * _M
    out0 = b * _N + loc0

    pltpu.sync_copy(xs_h.at[pl.ds(loc0, _QLOC)], xs_v)
    pltpu.sync_copy(ys_h.at[pl.ds(loc0, _QLOC)], ys_v)
    pltpu.sync_copy(llo_h.at[pl.ds(loc0, _QLOC)], llo_v)
    pltpu.sync_copy(lhi_h.at[pl.ds(loc0, _QLOC)], lhi_v)
    pltpu.sync_copy(rad_h.at[pl.ds(loc0, _QLOC)], rad_v)
    pltpu.sync_copy(bx1_h.at[pl.ds(box0, _M)], bx1_v)
    pltpu.sync_copy(by1_h.at[pl.ds(box0, _M)], by1_v)
    pltpu.sync_copy(bx2_h.at[pl.ds(box0, _M)], bx2_v)
    pltpu.sync_copy(by2_h.at[pl.ds(box0, _M)], by2_v)
    pltpu.sync_copy(lab_h.at[pl.ds(box0, _M)], lab_v)

    nchunk = _M // 16
    for k in range(nchunk):
        csl = pl.ds(k * 16, 16)
        x1ck = bx1_v[csl]
        y1ck = by1_v[csl]
        x2ck = bx2_v[csl]
        y2ck = by2_v[csl]
        cxck = (x1ck + x2ck) / 2.0
        cyck = (y1ck + y2ck) / 2.0
        for j in range(16):
            m = k * 16 + j
            msl = pl.ds(m * 16, 16)
            tx1_v[msl] = _splat(x1ck, j)
            ty1_v[msl] = _splat(y1ck, j)
            tx2_v[msl] = _splat(x2ck, j)
            ty2_v[msl] = _splat(y2ck, j)
            tcx_v[msl] = _splat(cxck, j)
            tcy_v[msl] = _splat(cyck, j)
            tidx_v[msl] = jnp.full((16,), m, jnp.int32)

    def group(gi, _):
        sls = [pl.ds(gi * 32, 16), pl.ds(gi * 32 + 16, 16)]
        xv = [xs_v[sl] for sl in sls]
        yv = [ys_v[sl] for sl in sls]
        llov = [llo_v[sl] for sl in sls]
        lhiv = [lhi_v[sl] for sl in sls]
        radv = [rad_v[sl] for sl in sls]

        barea = [jnp.full((16,), _BIG, jnp.float32) for _ in range(2)]
        bidx = [jnp.zeros((16,), jnp.int32) for _ in range(2)]
        for m in range(_M):
            msl = pl.ds(m * 16, 16)
            x1 = tx1_v[msl]
            y1 = ty1_v[msl]
            x2 = tx2_v[msl]
            y2 = ty2_v[msl]
            cx = tcx_v[msl]
            cy = tcy_v[msl]
            midx = tidx_v[msl]
            for u in range(2):
                l = xv[u] - x1
                t = yv[u] - y1
                r = x2 - xv[u]
                bb = y2 - yv[u]
                area = (l + r) * (t + bb)
                omin = jnp.minimum(jnp.minimum(l, t), jnp.minimum(r, bb))
                omax = jnp.maximum(jnp.maximum(l, t), jnp.maximum(r, bb))
                pos = (omin > 0.0) & (omax > llov[u]) & (omax <= lhiv[u])
                cd = jnp.maximum(jnp.abs(xv[u] - cx), jnp.abs(yv[u] - cy))
                pos = pos & (cd < radv[u])
                upd = pos & (area < barea[u])
                barea[u] = jnp.where(upd, area, barea[u])
                bidx[u] = jnp.where(upd, midx, bidx[u])

        x1c = [bx1_v[pl.ds(k * 16, 16)] for k in range(nchunk)]
        y1c = [by1_v[pl.ds(k * 16, 16)] for k in range(nchunk)]
        x2c = [bx2_v[pl.ds(k * 16, 16)] for k in range(nchunk)]
        y2c = [by2_v[pl.ds(k * 16, 16)] for k in range(nchunk)]
        labc = [lab_v[pl.ds(k * 16, 16)] for k in range(nchunk)]
        neg1 = jnp.full((16,), -1.0, jnp.float32)
        for u in range(2):
            sl = sls[u]
            anypos = barea[u] < 1e7
            il = bidx[u] & 15
            ksel = [bidx[u] >> 4 == k for k in range(1, nchunk)]

            def chunk_gather(arr):
                g = _dyn_gather(arr[0], il)
                for k in range(1, nchunk):
                    g = jnp.where(ksel[k - 1], _dyn_gather(arr[k], il), g)
                return g

            gx1 = chunk_gather(x1c)
            gy1 = chunk_gather(y1c)
            gx2 = chunk_gather(x2c)
            gy2 = chunk_gather(y2c)
            lab = chunk_gather(labc)
            l = xv[u] - gx1
            t = yv[u] - gy1
            r = gx2 - xv[u]
            bb = gy2 - yv[u]
            lrmin = jnp.minimum(l, r)
            lrmax = jnp.maximum(l, r)
            tbmin = jnp.minimum(t, bb)
            tbmax = jnp.maximum(t, bb)
            ratio = (lrmin * tbmin) / (lrmax * tbmax + 1e-10)
            ctr = jnp.where(anypos, _sqrt16(jnp.where(anypos, ratio, 1.0)),
                            -1.0)
            cls_v[sl] = jnp.where(anypos, lab, 0)
            ctr_v[sl] = ctr
            l_v[sl] = jnp.where(anypos, l, neg1)
            t_v[sl] = jnp.where(anypos, t, neg1)
            r_v[sl] = jnp.where(anypos, r, neg1)
            b_v[sl] = jnp.where(anypos, bb, neg1)
        return 0

    lax.fori_loop(0, _NG // 2, group, 0)

    @pl.when(q < 3)
    def _():
        pltpu.sync_copy(cls_v, cls_o.at[pl.ds(out0, _QLOC)])
        pltpu.sync_copy(ctr_v, ctr_o.at[pl.ds(out0, _QLOC)])
        pltpu.sync_copy(l_v, l_o.at[pl.ds(out0, _QLOC)])
        pltpu.sync_copy(t_v, t_o.at[pl.ds(out0, _QLOC)])
        pltpu.sync_copy(r_v, r_o.at[pl.ds(out0, _QLOC)])
        pltpu.sync_copy(b_v, b_o.at[pl.ds(out0, _QLOC)])

    @pl.when(q == 3)
    def _():
        nlast = _N - 3 * _QLOC
        pltpu.sync_copy(cls_v.at[pl.ds(0, nlast)],
                        cls_o.at[pl.ds(out0, nlast)])
        pltpu.sync_copy(ctr_v.at[pl.ds(0, nlast)],
                        ctr_o.at[pl.ds(out0, nlast)])
        pltpu.sync_copy(l_v.at[pl.ds(0, nlast)], l_o.at[pl.ds(out0, nlast)])
        pltpu.sync_copy(t_v.at[pl.ds(0, nlast)], t_o.at[pl.ds(out0, nlast)])
        pltpu.sync_copy(r_v.at[pl.ds(0, nlast)], r_o.at[pl.ds(out0, nlast)])
        pltpu.sync_copy(b_v.at[pl.ds(0, nlast)], b_o.at[pl.ds(out0, nlast)])


@jax.jit
def _gen_targets(gt_box, labels):
    bx1 = gt_box[..., 0].reshape(-1)
    by1 = gt_box[..., 1].reshape(-1)
    bx2 = gt_box[..., 2].reshape(-1)
    by2 = gt_box[..., 3].reshape(-1)
    lab = labels.astype(jnp.int32).reshape(-1)

    mesh = plsc.VectorSubcoreMesh(core_axis_name="c", subcore_axis_name="s")
    f32 = jnp.float32
    kfn = functools.partial(
        pl.kernel, mesh=mesh,
        out_type=[
            jax.ShapeDtypeStruct((_B * _N,), jnp.int32),
            jax.ShapeDtypeStruct((_B * _N,), f32),
            jax.ShapeDtypeStruct((_B * _N,), f32),
            jax.ShapeDtypeStruct((_B * _N,), f32),
            jax.ShapeDtypeStruct((_B * _N,), f32),
            jax.ShapeDtypeStruct((_B * _N,), f32),
        ],
        scratch_types=[
            pltpu.VMEM((_QLOC,), f32),
            pltpu.VMEM((_QLOC,), f32),
            pltpu.VMEM((_QLOC,), f32),
            pltpu.VMEM((_QLOC,), f32),
            pltpu.VMEM((_QLOC,), f32),
            pltpu.VMEM((_M,), f32),
            pltpu.VMEM((_M,), f32),
            pltpu.VMEM((_M,), f32),
            pltpu.VMEM((_M,), f32),
            pltpu.VMEM((_M,), jnp.int32),
            pltpu.VMEM((_M * 16,), f32),
            pltpu.VMEM((_M * 16,), f32),
            pltpu.VMEM((_M * 16,), f32),
            pltpu.VMEM((_M * 16,), f32),
            pltpu.VMEM((_M * 16,), f32),
            pltpu.VMEM((_M * 16,), f32),
            pltpu.VMEM((_M * 16,), jnp.int32),
            pltpu.VMEM((_QLOC,), jnp.int32),
            pltpu.VMEM((_QLOC,), f32),
            pltpu.VMEM((_QLOC,), f32),
            pltpu.VMEM((_QLOC,), f32),
            pltpu.VMEM((_QLOC,), f32),
            pltpu.VMEM((_QLOC,), f32),
        ],
    )(_sc_body)
    cls_p, ctr_p, l_p, t_p, r_p, b_p = kfn(
        jnp.asarray(_XS), jnp.asarray(_YS), jnp.asarray(_LLO),
        jnp.asarray(_LHI), jnp.asarray(_RAD), bx1, by1, bx2, by2, lab)
    cls_t = cls_p.reshape(_B, _N)[:, :, None]
    ctr_t = ctr_p.reshape(_B, _N)[:, :, None]
    reg_t = jnp.stack(
        [p.reshape(_B, _N) for p in (l_p, t_p, r_p, b_p)], axis=-1)
    return cls_t, ctr_t, reg_t


def kernel(cls_logit_0, cls_logit_1, cls_logit_2, cls_logit_3, cls_logit_4,
           center_logit_0, center_logit_1, center_logit_2, center_logit_3,
           center_logit_4, reg_logit_0, reg_logit_1, reg_logit_2,
           reg_logit_3, reg_logit_4, gt_box, labels):
    return _gen_targets(gt_box, labels)


# speedup vs baseline: 1.2565x; 1.2565x over previous
import functools
import numpy as np
import jax
import jax.numpy as jnp
from jax import lax
from jax.experimental import pallas as pl
from jax.experimental.pallas import tpu as pltpu
from jax.experimental.pallas import tpu_sc as plsc

_STRIDES = [8, 16, 32, 64, 128]
_LIMITS = [(-1.0, 64.0), (64.0, 128.0), (128.0, 256.0), (256.0, 512.0),
           (512.0, 99999999.0)]
_FEAT = [(64, 64), (32, 32), (16, 16), (8, 8), (4, 4)]
_B, _M = 8, 64
_N = sum(h * w for h, w in _FEAT)
_NPAD = 5504
_QLOC = _NPAD // 4
_NG = _QLOC // 16
_BIG = 99999999.0


def _location_tables():
    xs = np.zeros(_NPAD, np.float32)
    ys = np.zeros(_NPAD, np.float32)
    llo = np.full(_NPAD, 1e9, np.float32)
    lhi = np.full(_NPAD, -1e9, np.float32)
    rad = np.full(_NPAD, -1.0, np.float32)
    o = 0
    for (h, w), s, (lo, hi) in zip(_FEAT, _STRIDES, _LIMITS):
        sx = np.arange(0, w * s, s, dtype=np.float32) + s // 2
        sy = np.arange(0, h * s, s, dtype=np.float32) + s // 2
        yy, xx = np.meshgrid(sy, sx, indexing='ij')
        n = h * w
        xs[o:o + n] = xx.reshape(-1)
        ys[o:o + n] = yy.reshape(-1)
        llo[o:o + n] = lo
        lhi[o:o + n] = hi
        rad[o:o + n] = s * 1.5
        o += n
    return xs, ys, llo, lhi, rad


_XS, _YS, _LLO, _LHI, _RAD = _location_tables()

_NP = _NG // 2
_SEGOF = [
    [0] * _NP,
    [0] * _NP,
    [0] * 42 + [1],
    [0] * 31 + [1] * 8 + [2] * 2 + [3] * 2,
]
_SEGLV = [[0, 0, 0, 0], [0, 0, 0, 0], [0, 1, 1, 1], [1, 2, 3, 4]]
_SEGY = [
    [(4.0, 172.0), (0.0, 0.0), (0.0, 0.0), (0.0, 0.0)],
    [(172.0, 340.0), (0.0, 0.0), (0.0, 0.0), (0.0, 0.0)],
    [(348.0, 508.0), (8.0, 8.0), (0.0, 0.0), (0.0, 0.0)],
    [(24.0, 504.0), (16.0, 496.0), (32.0, 480.0), (64.0, 448.0)],
]
_LVRAD = [s * 1.5 for s in _STRIDES]
_LVXLO = [4.0, 8.0, 16.0, 32.0, 64.0]
_LVXHI = [508.0, 504.0, 496.0, 480.0, 448.0]


def _splat(v, j, dtype=jnp.float32):
    return jnp.full((16,), v[j], dtype)


def _dyn_gather(v, iv):
    return lax.gather(
        v, iv[:, None],
        dimension_numbers=lax.GatherDimensionNumbers(
            offset_dims=(), collapsed_slice_dims=(0,), start_index_map=(0,)),
        slice_sizes=(1,),
        mode=lax.GatherScatterMode.PROMISE_IN_BOUNDS)


def _sqrt16(x):
    i = lax.bitcast_convert_type(x, jnp.int32)
    y = lax.bitcast_convert_type(jnp.int32(0x5F3759DF) - (i >> 1), jnp.float32)
    for _ in range(3):
        y = y * (1.5 - 0.5 * x * y * y)
    return x * y


def _sc_body(xs_h, ys_h, llo_h, lhi_h, rad_h, bx1_h, by1_h, bx2_h, by2_h,
             lab_h, cls_o, ctr_o, l_o, t_o, r_o, b_o,
             xs_v, ys_v, llo_v, lhi_v, rad_v,
             bx1_v, by1_v, bx2_v, by2_v, lab_v,
             tx1_v, ty1_v, tx2_v, ty2_v, tcx_v, tcy_v,
             cls_v, ctr_v, l_v, t_v, r_v, b_v,
             seg_ref, idx_ref, cnt_ref):
    wid = lax.axis_index("s") * 2 + lax.axis_index("c")
    b = wid // 4
    q = wid % 4
    loc0 = q * _QLOC
    box0 = b * _M
    out0 = b * _N + loc0

    pltpu.sync_copy(xs_h.at[pl.ds(loc0, _QLOC)], xs_v)
    pltpu.sync_copy(ys_h.at[pl.ds(loc0, _QLOC)], ys_v)
    pltpu.sync_copy(llo_h.at[pl.ds(loc0, _QLOC)], llo_v)
    pltpu.sync_copy(lhi_h.at[pl.ds(loc0, _QLOC)], lhi_v)
    pltpu.sync_copy(rad_h.at[pl.ds(loc0, _QLOC)], rad_v)
    pltpu.sync_copy(bx1_h.at[pl.ds(box0, _M)], bx1_v)
    pltpu.sync_copy(by1_h.at[pl.ds(box0, _M)], by1_v)
    pltpu.sync_copy(bx2_h.at[pl.ds(box0, _M)], bx2_v)
    pltpu.sync_copy(by2_h.at[pl.ds(box0, _M)], by2_v)
    pltpu.sync_copy(lab_h.at[pl.ds(box0, _M)], lab_v)

    nchunk = _M // 16
    x1c = [bx1_v[pl.ds(k * 16, 16)] for k in range(nchunk)]
    y1c = [by1_v[pl.ds(k * 16, 16)] for k in range(nchunk)]
    x2c = [bx2_v[pl.ds(k * 16, 16)] for k in range(nchunk)]
    y2c = [by2_v[pl.ds(k * 16, 16)] for k in range(nchunk)]
    cxc = [(x1c[k] + x2c[k]) / 2.0 for k in range(nchunk)]
    cyc = [(y1c[k] + y2c[k]) / 2.0 for k in range(nchunk)]
    maxwh = [jnp.maximum(x2c[k] - x1c[k], y2c[k] - y1c[k])
             for k in range(nchunk)]

    for k in range(nchunk):
        for j in range(16):
            msl = pl.ds((k * 16 + j) * 16, 16)
            tx1_v[msl] = _splat(x1c[k], j)
            ty1_v[msl] = _splat(y1c[k], j)
            tx2_v[msl] = _splat(x2c[k], j)
            ty2_v[msl] = _splat(y2c[k], j)
            tcx_v[msl] = _splat(cxc[k], j)
            tcy_v[msl] = _splat(cyc[k], j)

    def qsel(vals, cast):
        r = cast(vals[3])
        for qq in (2, 1, 0):
            r = jnp.where(q == qq, cast(vals[qq]), r)
        return r

    for p in range(_NP):
        seg_ref[p] = qsel([_SEGOF[qq][p] for qq in range(4)], jnp.int32)

    for s in range(4):
        llo_s = qsel([_LIMITS[_SEGLV[qq][s]][0] for qq in range(4)],
                     jnp.float32)
        lhi_s = qsel([_LIMITS[_SEGLV[qq][s]][1] for qq in range(4)],
                     jnp.float32)
        rad_s = qsel([_LVRAD[_SEGLV[qq][s]] for qq in range(4)], jnp.float32)
        xlo_s = qsel([_LVXLO[_SEGLV[qq][s]] for qq in range(4)], jnp.float32)
        xhi_s = qsel([_LVXHI[_SEGLV[qq][s]] for qq in range(4)], jnp.float32)
        ylo_s = qsel([_SEGY[qq][s][0] for qq in range(4)], jnp.float32)
        yhi_s = qsel([_SEGY[qq][s][1] for qq in range(4)], jnp.float32)
        thr_lo = 2.0 * (llo_s - rad_s) - 1.0
        thr_hi = 2.0 * lhi_s + 1.0
        cnt_ref[s] = jnp.int32(0)
        for k in range(nchunk):
            keep = ((maxwh[k] > thr_lo) & (maxwh[k] < thr_hi)
                    & (cyc[k] > ylo_s - rad_s - 1.0)
                    & (cyc[k] < yhi_s + rad_s + 1.0)
                    & (y2c[k] > ylo_s - 1.0) & (y1c[k] < yhi_s + 1.0)
                    & (cxc[k] > xlo_s - rad_s - 1.0)
                    & (cxc[k] < xhi_s + rad_s + 1.0)
                    & (x2c[k] > xlo_s - 1.0) & (x1c[k] < xhi_s + 1.0))
            keepi = jnp.where(keep, 1, 0)
            for j in range(16):
                @pl.when(keepi[j] > 0)
                def _(k=k, j=j, s=s):
                    c = cnt_ref[s]
                    idx_ref[s * _M + c] = jnp.int32(k * 16 + j)
                    cnt_ref[s] = c + 1

    labc = [lab_v[pl.ds(k * 16, 16)] for k in range(nchunk)]
    neg1 = jnp.full((16,), -1.0, jnp.float32)

    def group(gi, _):
        sls = [pl.ds(gi * 32, 16), pl.ds(gi * 32 + 16, 16)]
        xv = [xs_v[sl] for sl in sls]
        yv = [ys_v[sl] for sl in sls]
        llov = [llo_v[sl] for sl in sls]
        lhiv = [lhi_v[sl] for sl in sls]
        radv = [rad_v[sl] for sl in sls]
        sid = seg_ref[gi]
        nbox = cnt_ref[sid]
        ibase = sid * _M

        def box(i, carry):
            ba0, bi0, ba1, bi1 = carry
            mm = idx_ref[ibase + i]
            msl = pl.ds(mm * 16, 16)
            x1 = tx1_v[msl]
            y1 = ty1_v[msl]
            x2 = tx2_v[msl]
            y2 = ty2_v[msl]
            cx = tcx_v[msl]
            cy = tcy_v[msl]
            midx = jnp.full((16,), mm, jnp.int32)
            out = []
            for u, (ba, bi) in enumerate(((ba0, bi0), (ba1, bi1))):
                l = xv[u] - x1
                t = yv[u] - y1
                r = x2 - xv[u]
                bb = y2 - yv[u]
                area = (l + r) * (t + bb)
                omin = jnp.minimum(jnp.minimum(l, t), jnp.minimum(r, bb))
                omax = jnp.maximum(jnp.maximum(l, t), jnp.maximum(r, bb))
                pos = (omin > 0.0) & (omax > llov[u]) & (omax <= lhiv[u])
                cd = jnp.maximum(jnp.abs(xv[u] - cx), jnp.abs(yv[u] - cy))
                pos = pos & (cd < radv[u])
                upd = pos & (area < ba)
                out.append((jnp.where(upd, area, ba),
                            jnp.where(upd, midx, bi)))
            return (out[0][0], out[0][1], out[1][0], out[1][1])

        init = (jnp.full((16,), _BIG, jnp.float32),
                jnp.zeros((16,), jnp.int32),
                jnp.full((16,), _BIG, jnp.float32),
                jnp.zeros((16,), jnp.int32))
        ba0, bi0, ba1, bi1 = lax.fori_loop(0, nbox, box, init)
        barea = [ba0, ba1]
        bidx = [bi0, bi1]
        for u in range(2):
            sl = sls[u]
            anypos = barea[u] < 1e7
            il = bidx[u] & 15
            ksel = [bidx[u] >> 4 == k for k in range(1, nchunk)]

            def chunk_gather(arr):
                g = _dyn_gather(arr[0], il)
                for k in range(1, nchunk):
                    g = jnp.where(ksel[k - 1], _dyn_gather(arr[k], il), g)
                return g

            gx1 = chunk_gather(x1c)
            gy1 = chunk_gather(y1c)
            gx2 = chunk_gather(x2c)
            gy2 = chunk_gather(y2c)
            lab = chunk_gather(labc)
            l = xv[u] - gx1
            t = yv[u] - gy1
            r = gx2 - xv[u]
            bb = gy2 - yv[u]
            lrmin = jnp.minimum(l, r)
            lrmax = jnp.maximum(l, r)
            tbmin = jnp.minimum(t, bb)
            tbmax = jnp.maximum(t, bb)
            ratio = (lrmin * tbmin) / (lrmax * tbmax + 1e-10)
            ctr = jnp.where(anypos, _sqrt16(jnp.where(anypos, ratio, 1.0)),
                            -1.0)
            cls_v[sl] = jnp.where(anypos, lab, 0)
            ctr_v[sl] = ctr
            l_v[sl] = jnp.where(anypos, l, neg1)
            t_v[sl] = jnp.where(anypos, t, neg1)
            r_v[sl] = jnp.where(anypos, r, neg1)
            b_v[sl] = jnp.where(anypos, bb, neg1)
        return 0

    lax.fori_loop(0, _NP, group, 0)

    @pl.when(q < 3)
    def _():
        pltpu.sync_copy(cls_v, cls_o.at[pl.ds(out0, _QLOC)])
        pltpu.sync_copy(ctr_v, ctr_o.at[pl.ds(out0, _QLOC)])
        pltpu.sync_copy(l_v, l_o.at[pl.ds(out0, _QLOC)])
        pltpu.sync_copy(t_v, t_o.at[pl.ds(out0, _QLOC)])
        pltpu.sync_copy(r_v, r_o.at[pl.ds(out0, _QLOC)])
        pltpu.sync_copy(b_v, b_o.at[pl.ds(out0, _QLOC)])

    @pl.when(q == 3)
    def _():
        nlast = _N - 3 * _QLOC
        pltpu.sync_copy(cls_v.at[pl.ds(0, nlast)],
                        cls_o.at[pl.ds(out0, nlast)])
        pltpu.sync_copy(ctr_v.at[pl.ds(0, nlast)],
                        ctr_o.at[pl.ds(out0, nlast)])
        pltpu.sync_copy(l_v.at[pl.ds(0, nlast)], l_o.at[pl.ds(out0, nlast)])
        pltpu.sync_copy(t_v.at[pl.ds(0, nlast)], t_o.at[pl.ds(out0, nlast)])
        pltpu.sync_copy(r_v.at[pl.ds(0, nlast)], r_o.at[pl.ds(out0, nlast)])
        pltpu.sync_copy(b_v.at[pl.ds(0, nlast)], b_o.at[pl.ds(out0, nlast)])


@jax.jit
def _gen_targets(gt_box, labels):
    bx1 = gt_box[..., 0].reshape(-1)
    by1 = gt_box[..., 1].reshape(-1)
    bx2 = gt_box[..., 2].reshape(-1)
    by2 = gt_box[..., 3].reshape(-1)
    lab = labels.astype(jnp.int32).reshape(-1)

    mesh = plsc.VectorSubcoreMesh(core_axis_name="c", subcore_axis_name="s")
    f32 = jnp.float32
    kfn = functools.partial(
        pl.kernel, mesh=mesh,
        out_type=[
            jax.ShapeDtypeStruct((_B * _N,), jnp.int32),
            jax.ShapeDtypeStruct((_B * _N,), f32),
            jax.ShapeDtypeStruct((_B * _N,), f32),
            jax.ShapeDtypeStruct((_B * _N,), f32),
            jax.ShapeDtypeStruct((_B * _N,), f32),
            jax.ShapeDtypeStruct((_B * _N,), f32),
        ],
        scratch_types=[
            pltpu.VMEM((_QLOC,), f32),
            pltpu.VMEM((_QLOC,), f32),
            pltpu.VMEM((_QLOC,), f32),
            pltpu.VMEM((_QLOC,), f32),
            pltpu.VMEM((_QLOC,), f32),
            pltpu.VMEM((_M,), f32),
            pltpu.VMEM((_M,), f32),
            pltpu.VMEM((_M,), f32),
            pltpu.VMEM((_M,), f32),
            pltpu.VMEM((_M,), jnp.int32),
            pltpu.VMEM((_M * 16,), f32),
            pltpu.VMEM((_M * 16,), f32),
            pltpu.VMEM((_M * 16,), f32),
            pltpu.VMEM((_M * 16,), f32),
            pltpu.VMEM((_M * 16,), f32),
            pltpu.VMEM((_M * 16,), f32),
            pltpu.VMEM((_QLOC,), jnp.int32),
            pltpu.VMEM((_QLOC,), f32),
            pltpu.VMEM((_QLOC,), f32),
            pltpu.VMEM((_QLOC,), f32),
            pltpu.VMEM((_QLOC,), f32),
            pltpu.VMEM((_QLOC,), f32),
            pltpu.SMEM((_NP,), jnp.int32),
            pltpu.SMEM((4 * _M,), jnp.int32),
            pltpu.SMEM((4,), jnp.int32),
        ],
    )(_sc_body)
    cls_p, ctr_p, l_p, t_p, r_p, b_p = kfn(
        jnp.asarray(_XS), jnp.asarray(_YS), jnp.asarray(_LLO),
        jnp.asarray(_LHI), jnp.asarray(_RAD), bx1, by1, bx2, by2, lab)
    cls_t = cls_p.reshape(_B, _N)[:, :, None]
    ctr_t = ctr_p.reshape(_B, _N)[:, :, None]
    reg_t = jnp.stack(
        [p.reshape(_B, _N) for p in (l_p, t_p, r_p, b_p)], axis=-1)
    return cls_t, ctr_t, reg_t


def kernel(cls_logit_0, cls_logit_1, cls_logit_2, cls_logit_3, cls_logit_4,
           center_logit_0, center_logit_1, center_logit_2, center_logit_3,
           center_logit_4, reg_logit_0, reg_logit_1, reg_logit_2,
           reg_logit_3, reg_logit_4, gt_box, labels):
    return _gen_targets(gt_box, labels)


# speedup vs baseline: 1.3979x; 1.1125x over previous
import functools
import numpy as np
import jax
import jax.numpy as jnp
from jax import lax
from jax.experimental import pallas as pl
from jax.experimental.pallas import tpu as pltpu
from jax.experimental.pallas import tpu_sc as plsc

_STRIDES = [8, 16, 32, 64, 128]
_LIMITS = [(-1.0, 64.0), (64.0, 128.0), (128.0, 256.0), (256.0, 512.0),
           (512.0, 99999999.0)]
_FEAT = [(64, 64), (32, 32), (16, 16), (8, 8), (4, 4)]
_B, _M = 8, 64
_N = sum(h * w for h, w in _FEAT)
_WSPAN = 1920
_NIN = 4864 + _WSPAN
_BIG = 99999999.0


def _location_tables():
    xs = np.zeros(_NIN, np.float32)
    ys = np.zeros(_NIN, np.float32)
    llo = np.full(_NIN, 1e9, np.float32)
    lhi = np.full(_NIN, -1e9, np.float32)
    rad = np.full(_NIN, -1.0, np.float32)
    o = 0
    for (h, w), s, (lo, hi) in zip(_FEAT, _STRIDES, _LIMITS):
        sx = np.arange(0, w * s, s, dtype=np.float32) + s // 2
        sy = np.arange(0, h * s, s, dtype=np.float32) + s // 2
        yy, xx = np.meshgrid(sy, sx, indexing='ij')
        n = h * w
        xs[o:o + n] = xx.reshape(-1)
        ys[o:o + n] = yy.reshape(-1)
        llo[o:o + n] = lo
        lhi[o:o + n] = hi
        rad[o:o + n] = s * 1.5
        o += n
    return xs, ys, llo, lhi, rad


_XS, _YS, _LLO, _LHI, _RAD = _location_tables()

_NPMAX = 60
_QLOC0 = [0, 1920, 3840, 4864]
_QNP = [60, 60, 32, 20]
_QOUT = [1920, 1920, 1024, 592]
_SEGOF = [
    [0] * 60,
    [0] * 60,
    [0] * 8 + [1] * 24 + [3] * 28,
    [0] * 8 + [1] * 8 + [2] * 2 + [3] * 42,
]
_SEGLV = [[0, 0, 0, 0], [0, 0, 0, 0], [0, 1, 1, 1], [1, 2, 3, 4]]
_SEGY = [
    [(4.0, 236.0), (0.0, 0.0), (0.0, 0.0), (0.0, 0.0)],
    [(244.0, 476.0), (0.0, 0.0), (0.0, 0.0), (0.0, 0.0)],
    [(484.0, 508.0), (8.0, 376.0), (0.0, 0.0), (0.0, 0.0)],
    [(392.0, 504.0), (16.0, 496.0), (32.0, 480.0), (64.0, 448.0)],
]
_LVRAD = [s * 1.5 for s in _STRIDES]
_LVXLO = [4.0, 8.0, 16.0, 32.0, 64.0]
_LVXHI = [508.0, 504.0, 496.0, 480.0, 448.0]


def _splat(v, j, dtype=jnp.float32):
    return jnp.full((16,), v[j], dtype)


def _dyn_gather(v, iv):
    return lax.gather(
        v, iv[:, None],
        dimension_numbers=lax.GatherDimensionNumbers(
            offset_dims=(), collapsed_slice_dims=(0,), start_index_map=(0,)),
        slice_sizes=(1,),
        mode=lax.GatherScatterMode.PROMISE_IN_BOUNDS)


def _sqrt16(x):
    i = lax.bitcast_convert_type(x, jnp.int32)
    y = lax.bitcast_convert_type(jnp.int32(0x5F3759DF) - (i >> 1), jnp.float32)
    for _ in range(3):
        y = y * (1.5 - 0.5 * x * y * y)
    return x * y


def _sc_body(xs_h, ys_h, llo_h, lhi_h, rad_h, bx1_h, by1_h, bx2_h, by2_h,
             lab_h, cls_o, ctr_o, l_o, t_o, r_o, b_o,
             xs_v, ys_v, llo_v, lhi_v, rad_v,
             bx1_v, by1_v, bx2_v, by2_v, lab_v,
             tx1_v, ty1_v, tx2_v, ty2_v, tcx_v, tcy_v,
             cls_v, ctr_v, l_v, t_v, r_v, b_v,
             seg_ref, idx_ref, cnt_ref):
    wid = lax.axis_index("s") * 2 + lax.axis_index("c")
    b = wid // 4
    q = wid % 4
    box0 = b * _M

    def qsel(vals, cast):
        r = cast(vals[3])
        for qq in (2, 1, 0):
            r = jnp.where(q == qq, cast(vals[qq]), r)
        return r

    loc0 = qsel(_QLOC0, jnp.int32)
    out0 = b * _N + loc0

    pltpu.sync_copy(xs_h.at[pl.ds(loc0, _WSPAN)], xs_v)
    pltpu.sync_copy(ys_h.at[pl.ds(loc0, _WSPAN)], ys_v)
    pltpu.sync_copy(llo_h.at[pl.ds(loc0, _WSPAN)], llo_v)
    pltpu.sync_copy(lhi_h.at[pl.ds(loc0, _WSPAN)], lhi_v)
    pltpu.sync_copy(rad_h.at[pl.ds(loc0, _WSPAN)], rad_v)
    pltpu.sync_copy(bx1_h.at[pl.ds(box0, _M)], bx1_v)
    pltpu.sync_copy(by1_h.at[pl.ds(box0, _M)], by1_v)
    pltpu.sync_copy(bx2_h.at[pl.ds(box0, _M)], bx2_v)
    pltpu.sync_copy(by2_h.at[pl.ds(box0, _M)], by2_v)
    pltpu.sync_copy(lab_h.at[pl.ds(box0, _M)], lab_v)

    nchunk = _M // 16
    x1c = [bx1_v[pl.ds(k * 16, 16)] for k in range(nchunk)]
    y1c = [by1_v[pl.ds(k * 16, 16)] for k in range(nchunk)]
    x2c = [bx2_v[pl.ds(k * 16, 16)] for k in range(nchunk)]
    y2c = [by2_v[pl.ds(k * 16, 16)] for k in range(nchunk)]
    cxc = [(x1c[k] + x2c[k]) / 2.0 for k in range(nchunk)]
    cyc = [(y1c[k] + y2c[k]) / 2.0 for k in range(nchunk)]
    maxwh = [jnp.maximum(x2c[k] - x1c[k], y2c[k] - y1c[k])
             for k in range(nchunk)]

    for k in range(nchunk):
        for j in range(16):
            msl = pl.ds((k * 16 + j) * 16, 16)
            tx1_v[msl] = _splat(x1c[k], j)
            ty1_v[msl] = _splat(y1c[k], j)
            tx2_v[msl] = _splat(x2c[k], j)
            ty2_v[msl] = _splat(y2c[k], j)
            tcx_v[msl] = _splat(cxc[k], j)
            tcy_v[msl] = _splat(cyc[k], j)

    for p in range(_NPMAX):
        seg_ref[p] = qsel([_SEGOF[qq][p] for qq in range(4)], jnp.int32)

    for s in range(4):
        llo_s = qsel([_LIMITS[_SEGLV[qq][s]][0] for qq in range(4)],
                     jnp.float32)
        lhi_s = qsel([_LIMITS[_SEGLV[qq][s]][1] for qq in range(4)],
                     jnp.float32)
        rad_s = qsel([_LVRAD[_SEGLV[qq][s]] for qq in range(4)], jnp.float32)
        xlo_s = qsel([_LVXLO[_SEGLV[qq][s]] for qq in range(4)], jnp.float32)
        xhi_s = qsel([_LVXHI[_SEGLV[qq][s]] for qq in range(4)], jnp.float32)
        ylo_s = qsel([_SEGY[qq][s][0] for qq in range(4)], jnp.float32)
        yhi_s = qsel([_SEGY[qq][s][1] for qq in range(4)], jnp.float32)
        thr_lo = 2.0 * (llo_s - rad_s) - 1.0
        thr_hi = 2.0 * lhi_s + 1.0
        cnt_ref[s] = jnp.int32(0)
        for k in range(nchunk):
            keep = ((maxwh[k] > thr_lo) & (maxwh[k] < thr_hi)
                    & (cyc[k] > ylo_s - rad_s - 1.0)
                    & (cyc[k] < yhi_s + rad_s + 1.0)
                    & (y2c[k] > ylo_s - 1.0) & (y1c[k] < yhi_s + 1.0)
                    & (cxc[k] > xlo_s - rad_s - 1.0)
                    & (cxc[k] < xhi_s + rad_s + 1.0)
                    & (x2c[k] > xlo_s - 1.0) & (x1c[k] < xhi_s + 1.0))
            keepi = jnp.where(keep, 1, 0)
            for j in range(16):
                @pl.when(keepi[j] > 0)
                def _(k=k, j=j, s=s):
                    c = cnt_ref[s]
                    idx_ref[s * _M + c] = jnp.int32(k * 16 + j)
                    cnt_ref[s] = c + 1

    labc = [lab_v[pl.ds(k * 16, 16)] for k in range(nchunk)]
    neg1 = jnp.full((16,), -1.0, jnp.float32)

    def group(gi, _):
        sls = [pl.ds(gi * 32, 16), pl.ds(gi * 32 + 16, 16)]
        xv = [xs_v[sl] for sl in sls]
        yv = [ys_v[sl] for sl in sls]
        llov = [llo_v[sl] for sl in sls]
        lhiv = [lhi_v[sl] for sl in sls]
        radv = [rad_v[sl] for sl in sls]
        sid = seg_ref[gi]
        nbox = cnt_ref[sid]
        ibase = sid * _M

        def box(i, carry):
            ba0, bi0, ba1, bi1 = carry
            mm = idx_ref[ibase + i]
            msl = pl.ds(mm * 16, 16)
            x1 = tx1_v[msl]
            y1 = ty1_v[msl]
            x2 = tx2_v[msl]
            y2 = ty2_v[msl]
            cx = tcx_v[msl]
            cy = tcy_v[msl]
            midx = jnp.full((16,), mm, jnp.int32)
            out = []
            for u, (ba, bi) in enumerate(((ba0, bi0), (ba1, bi1))):
                l = xv[u] - x1
                t = yv[u] - y1
                r = x2 - xv[u]
                bb = y2 - yv[u]
                area = (l + r) * (t + bb)
                omin = jnp.minimum(jnp.minimum(l, t), jnp.minimum(r, bb))
                omax = jnp.maximum(jnp.maximum(l, t), jnp.maximum(r, bb))
                pos = (omin > 0.0) & (omax > llov[u]) & (omax <= lhiv[u])
                cd = jnp.maximum(jnp.abs(xv[u] - cx), jnp.abs(yv[u] - cy))
                pos = pos & (cd < radv[u])
                upd = pos & (area < ba)
                out.append((jnp.where(upd, area, ba),
                            jnp.where(upd, midx, bi)))
            return (out[0][0], out[0][1], out[1][0], out[1][1])

        init = (jnp.full((16,), _BIG, jnp.float32),
                jnp.zeros((16,), jnp.int32),
                jnp.full((16,), _BIG, jnp.float32),
                jnp.zeros((16,), jnp.int32))
        ba0, bi0, ba1, bi1 = lax.fori_loop(0, nbox, box, init)
        barea = [ba0, ba1]
        bidx = [bi0, bi1]
        for u in range(2):
            sl = sls[u]
            anypos = barea[u] < 1e7
            il = bidx[u] & 15
            ksel = [bidx[u] >> 4 == k for k in range(1, nchunk)]

            def chunk_gather(arr):
                g = _dyn_gather(arr[0], il)
                for k in range(1, nchunk):
                    g = jnp.where(ksel[k - 1], _dyn_gather(arr[k], il), g)
                return g

            gx1 = chunk_gather(x1c)
            gy1 = chunk_gather(y1c)
            gx2 = chunk_gather(x2c)
            gy2 = chunk_gather(y2c)
            lab = chunk_gather(labc)
            l = xv[u] - gx1
            t = yv[u] - gy1
            r = gx2 - xv[u]
            bb = gy2 - yv[u]
            lrmin = jnp.minimum(l, r)
            lrmax = jnp.maximum(l, r)
            tbmin = jnp.minimum(t, bb)
            tbmax = jnp.maximum(t, bb)
            ratio = (lrmin * tbmin) / (lrmax * tbmax + 1e-10)
            ctr = jnp.where(anypos, _sqrt16(jnp.where(anypos, ratio, 1.0)),
                            -1.0)
            cls_v[sl] = jnp.where(anypos, lab, 0)
            ctr_v[sl] = ctr
            l_v[sl] = jnp.where(anypos, l, neg1)
            t_v[sl] = jnp.where(anypos, t, neg1)
            r_v[sl] = jnp.where(anypos, r, neg1)
            b_v[sl] = jnp.where(anypos, bb, neg1)
        return 0

    lax.fori_loop(0, qsel(_QNP, jnp.int32), group, 0)

    for qq in range(4):
        @pl.when(q == qq)
        def _(qq=qq):
            nout = _QOUT[qq]
            pltpu.sync_copy(cls_v.at[pl.ds(0, nout)],
                            cls_o.at[pl.ds(out0, nout)])
            pltpu.sync_copy(ctr_v.at[pl.ds(0, nout)],
                            ctr_o.at[pl.ds(out0, nout)])
            pltpu.sync_copy(l_v.at[pl.ds(0, nout)],
                            l_o.at[pl.ds(out0, nout)])
            pltpu.sync_copy(t_v.at[pl.ds(0, nout)],
                            t_o.at[pl.ds(out0, nout)])
            pltpu.sync_copy(r_v.at[pl.ds(0, nout)],
                            r_o.at[pl.ds(out0, nout)])
            pltpu.sync_copy(b_v.at[pl.ds(0, nout)],
                            b_o.at[pl.ds(out0, nout)])


@jax.jit
def _gen_targets(gt_box, labels):
    bx1 = gt_box[..., 0].reshape(-1)
    by1 = gt_box[..., 1].reshape(-1)
    bx2 = gt_box[..., 2].reshape(-1)
    by2 = gt_box[..., 3].reshape(-1)
    lab = labels.astype(jnp.int32).reshape(-1)

    mesh = plsc.VectorSubcoreMesh(core_axis_name="c", subcore_axis_name="s")
    f32 = jnp.float32
    kfn = functools.partial(
        pl.kernel, mesh=mesh,
        out_type=[
            jax.ShapeDtypeStruct((_B * _N,), jnp.int32),
            jax.ShapeDtypeStruct((_B * _N,), f32),
            jax.ShapeDtypeStruct((_B * _N,), f32),
            jax.ShapeDtypeStruct((_B * _N,), f32),
            jax.ShapeDtypeStruct((_B * _N,), f32),
            jax.ShapeDtypeStruct((_B * _N,), f32),
        ],
        scratch_types=[
            pltpu.VMEM((_WSPAN,), f32),
            pltpu.VMEM((_WSPAN,), f32),
            pltpu.VMEM((_WSPAN,), f32),
            pltpu.VMEM((_WSPAN,), f32),
            pltpu.VMEM((_WSPAN,), f32),
            pltpu.VMEM((_M,), f32),
            pltpu.VMEM((_M,), f32),
            pltpu.VMEM((_M,), f32),
            pltpu.VMEM((_M,), f32),
            pltpu.VMEM((_M,), jnp.int32),
            pltpu.VMEM((_M * 16,), f32),
            pltpu.VMEM((_M * 16,), f32),
            pltpu.VMEM((_M * 16,), f32),
            pltpu.VMEM((_M * 16,), f32),
            pltpu.VMEM((_M * 16,), f32),
            pltpu.VMEM((_M * 16,), f32),
            pltpu.VMEM((_WSPAN,), jnp.int32),
            pltpu.VMEM((_WSPAN,), f32),
            pltpu.VMEM((_WSPAN,), f32),
            pltpu.VMEM((_WSPAN,), f32),
            pltpu.VMEM((_WSPAN,), f32),
            pltpu.VMEM((_WSPAN,), f32),
            pltpu.SMEM((_NPMAX,), jnp.int32),
            pltpu.SMEM((4 * _M,), jnp.int32),
            pltpu.SMEM((4,), jnp.int32),
        ],
    )(_sc_body)
    cls_p, ctr_p, l_p, t_p, r_p, b_p = kfn(
        jnp.asarray(_XS), jnp.asarray(_YS), jnp.asarray(_LLO),
        jnp.asarray(_LHI), jnp.asarray(_RAD), bx1, by1, bx2, by2, lab)
    cls_t = cls_p.reshape(_B, _N)[:, :, None]
    ctr_t = ctr_p.reshape(_B, _N)[:, :, None]
    reg_t = jnp.stack(
        [p.reshape(_B, _N) for p in (l_p, t_p, r_p, b_p)], axis=-1)
    return cls_t, ctr_t, reg_t


def kernel(cls_logit_0, cls_logit_1, cls_logit_2, cls_logit_3, cls_logit_4,
           center_logit_0, center_logit_1, center_logit_2, center_logit_3,
           center_logit_4, reg_logit_0, reg_logit_1, reg_logit_2,
           reg_logit_3, reg_logit_4, gt_box, labels):
    return _gen_targets(gt_box, labels)
